# baseline (speedup 1.0000x reference)
import functools

import jax
import jax.numpy as jnp
from jax import lax
from jax.experimental import pallas as pl
from jax.experimental.pallas import tpu as pltpu
from jax.experimental.pallas import tpu_sc as plsc

_VOCAB = 1000000
_DIM = 32
_BATCH = 16384

_NC = 2
_NS = 16
_L = 16
_NW = _NC * _NS
_B_PER_W = _BATCH // _NW
_NG = _B_PER_W // _L

_mesh = plsc.VectorSubcoreMesh(core_axis_name="c", subcore_axis_name="s")


@functools.partial(
    pl.kernel,
    out_type=jax.ShapeDtypeStruct((_BATCH, _DIM), jnp.float32),
    mesh=_mesh,
    scratch_types=[
        pltpu.VMEM((_B_PER_W,), jnp.int32),
        pltpu.VMEM((_L, 8, _DIM), jnp.float32),
        pltpu.VMEM((_B_PER_W, _DIM), jnp.float32),
        pltpu.SemaphoreType.DMA,
    ],
    compiler_params=pltpu.CompilerParams(
        use_tc_tiling_on_sc=True, needs_layout_passes=False
    ),
)
def _sc_embed(idx_hbm, table_hbm, out_hbm, idx_v, blk_v, out_v, sem):
    wid = lax.axis_index("s") * _NC + lax.axis_index("c")
    base = wid * _B_PER_W
    pltpu.sync_copy(idx_hbm.at[pl.ds(base, _B_PER_W)], idx_v)

    lane = lax.iota(jnp.int32, _L)

    def group(g):
        iv = idx_v[pl.ds(g * _L, _L)]
        handles = []
        for e in range(_L):
            r = iv[e]
            t8 = pl.multiple_of((r // 8) * 8, 8)
            handles.append(
                pltpu.async_copy(table_hbm.at[pl.ds(t8, 8), :], blk_v.at[e], sem)
            )
        for h in handles:
            h.wait()
        svec = lax.rem(iv, jnp.full((_L,), 8, jnp.int32))
        orow = g * _L + lane
        for d in range(_DIM):
            dvec = jnp.full((_L,), d, jnp.int32)
            vals = plsc.load_gather(blk_v, [lane, svec, dvec])
            plsc.store_scatter(out_v, [orow, dvec], vals)

    pl.loop(0, _NG)(group)
    pltpu.sync_copy(out_v, out_hbm.at[pl.ds(base, _B_PER_W)])


@jax.jit
def kernel(action, table):
    idx = action.astype(jnp.int32)
    return _sc_embed(idx, table)


# speedup vs baseline: 2.5100x; 2.5100x over previous
import functools

import jax
import jax.numpy as jnp
from jax import lax
from jax.experimental import pallas as pl
from jax.experimental.pallas import tpu as pltpu
from jax.experimental.pallas import tpu_sc as plsc

_VOCAB = 1000000
_DIM = 32
_BATCH = 16384

_NC = 2
_NS = 16
_L = 16
_NW = _NC * _NS
_B_PER_W = _BATCH // _NW
_NG = _B_PER_W // _L
_HALF = _B_PER_W // 2

_mesh = plsc.VectorSubcoreMesh(core_axis_name="c", subcore_axis_name="s")


@functools.partial(
    pl.kernel,
    out_type=jax.ShapeDtypeStruct((_BATCH, _DIM), jnp.float32),
    mesh=_mesh,
    scratch_types=[
        pltpu.VMEM((_B_PER_W,), jnp.int32),
        pltpu.VMEM((_L, _DIM, 128), jnp.float32),
        pltpu.VMEM((_HALF, _DIM), jnp.float32),
        pltpu.SemaphoreType.DMA,
    ],
    compiler_params=pltpu.CompilerParams(
        use_tc_tiling_on_sc=True, needs_layout_passes=False
    ),
)
def _sc_embed(idx_hbm, tt_hbm, out_hbm, idx_v, blk_v, out_v, sem):
    wid = lax.axis_index("s") * _NC + lax.axis_index("c")
    base = wid * _B_PER_W
    pltpu.sync_copy(idx_hbm.at[pl.ds(base, _B_PER_W)], idx_v)

    lane = lax.iota(jnp.int32, _L)

    def half(h):
        def group(g):
            gg = h * (_NG // 2) + g
            iv = idx_v[pl.ds(gg * _L, _L)]
            handles = []
            for e in range(_L):
                r = iv[e]
                c128 = pl.multiple_of((r // 128) * 128, 128)
                handles.append(
                    pltpu.async_copy(
                        tt_hbm.at[:, pl.ds(c128, 128)], blk_v.at[e], sem
                    )
                )
            for h2 in handles:
                h2.wait()
            lvec = lax.rem(iv, jnp.full((_L,), 128, jnp.int32))
            orow = g * _L + lane
            for d in range(_DIM):
                dvec = jnp.full((_L,), d, jnp.int32)
                vals = plsc.load_gather(blk_v, [lane, dvec, lvec])
                plsc.store_scatter(out_v, [orow, dvec], vals)

        pl.loop(0, _NG // 2)(group)
        pltpu.sync_copy(out_v, out_hbm.at[pl.ds(base + h * _HALF, _HALF)])

    half(0)
    half(1)


@jax.jit
def kernel(action, table):
    idx = action.astype(jnp.int32)
    return _sc_embed(idx, table.T)


# speedup vs baseline: 2.6034x; 1.0372x over previous
import functools

import jax
import jax.numpy as jnp
from jax import lax
from jax.experimental import pallas as pl
from jax.experimental.pallas import tpu as pltpu
from jax.experimental.pallas import tpu_sc as plsc

VOCAB, DIM, BATCH = 1000000, 32, 16384
NC, NS, L = 2, 16, 16
NW = NC * NS
BPW = BATCH // NW
GSZ = 8
HALF = BPW // 2
NGH = HALF // GSZ

sc_mesh = plsc.VectorSubcoreMesh(core_axis_name="c", subcore_axis_name="s")

@functools.partial(
    pl.kernel,
    out_type=jax.ShapeDtypeStruct((BATCH, DIM), jnp.float32),
    mesh=sc_mesh,
    scratch_types=[
        pltpu.VMEM((BPW + L,), jnp.int32),
        pltpu.VMEM((2 * GSZ, DIM, 128), jnp.float32),
        pltpu.VMEM((HALF, DIM), jnp.float32),
        pltpu.SemaphoreType.DMA,
        pltpu.SemaphoreType.DMA,
    ],
    compiler_params=pltpu.CompilerParams(
        use_tc_tiling_on_sc=True, needs_layout_passes=False
    ),
)
def sc_embed(idx_hbm, tt_hbm, out_hbm, idx_v, blk_v, out_v, sem0, sem1):
    wid = lax.axis_index("s") * NC + lax.axis_index("c")
    base = wid * BPW
    pltpu.sync_copy(idx_hbm.at[pl.ds(base, BPW)], idx_v.at[pl.ds(0, BPW)])

    lane = lax.iota(jnp.int32, L)
    emask = lane < GSZ

    def fire(goff, ring_half, sem):
        iv = idx_v[pl.ds(goff * GSZ, L)]
        for e in range(GSZ):
            r = iv[e]
            c128 = pl.multiple_of((r // 128) * 128, 128)
            pltpu.async_copy(
                tt_hbm.at[:, pl.ds(c128, 128)],
                blk_v.at[ring_half * GSZ + e],
                sem,
            )

    def drain(ring_half, sem):
        for e in range(GSZ):
            pltpu.make_async_copy(
                tt_hbm.at[:, pl.ds(0, 128)],
                blk_v.at[ring_half * GSZ + e],
                sem,
            ).wait()

    def half(h):
        hoff = h * NGH

        fire(hoff, 0, sem0)

        def pair(p):
            g0 = 2 * p
            g1 = 2 * p + 1

            @pl.when(g1 < NGH)
            def _():
                fire(hoff + g1, 1, sem1)

            drain(0, sem0)
            extract(hoff + g0, 0, g0)

            @pl.when(g1 + 1 < NGH)
            def _():
                fire(hoff + g1 + 1, 0, sem0)

            @pl.when(g1 < NGH)
            def _():
                drain(1, sem1)
                extract(hoff + g1, 1, g1)

        def extract(gabs, ring_half, glocal):
            iv = idx_v[pl.ds(gabs * GSZ, L)]
            lvec = lax.rem(iv, jnp.full((L,), 128, jnp.int32))
            slot = lane + ring_half * GSZ
            orow = glocal * GSZ + lane
            for d in range(DIM):
                dvec = jnp.full((L,), d, jnp.int32)
                vals = plsc.load_gather(blk_v, [slot, dvec, lvec])
                plsc.store_scatter(out_v, [orow, dvec], vals, mask=emask)

        pl.loop(0, NGH // 2)(pair)
        pltpu.sync_copy(out_v, out_hbm.at[pl.ds(base + h * HALF, HALF)])

    half(0)
    half(1)


@jax.jit
def kernel(action, table):
    idx = action.astype(jnp.int32)
    return sc_embed(idx, table.T)
